# Initial kernel scaffold; baseline (speedup 1.0000x reference)
import jax
import jax.numpy as jnp
from jax import lax
from jax.experimental import pallas as pl
from jax.experimental.pallas import tpu as pltpu
from jax.experimental.pallas import tpu_sc as plsc

N = 50000
E = 800000
D = 64
L = 6
CD = 2

F = 80
VW = 68

NC = 2
NS = 16
NW = NC * NS

HALF = N // 2
ACC_PAD = 1000
ACC_ROWS = HALF + ACC_PAD
TRASH = HALF
ZR = ACC_ROWS // NS

E_PER_W = E // NW
GCH = 1000
G_ITERS = E_PER_W // GCH

E_PER_S = E // NS
SCH = 1000
S_ITERS = E_PER_S // SCH

BE = 4000
NBLK = E // BE
RN = 1000
RB = 2000


def _silu(v):
    return v * jax.nn.sigmoid(v)


def _ln(v, g, b):
    m = v.mean(axis=-1, keepdims=True)
    var = ((v - m) ** 2).mean(axis=-1, keepdims=True)
    return (v - m) / jnp.sqrt(var + 1e-5) * g + b


_sc_mesh = plsc.VectorSubcoreMesh(
    core_axis_name="c", subcore_axis_name="s", num_cores=NC, num_subcores=NS)


def _gather_body(t_hbm, src_hbm, dst_hbm, gsrc_hbm, gdst_hbm,
                 idx_v, rows_v, idx2_v, rows2_v, sem, sem2):
    wid = lax.axis_index("c") * NS + lax.axis_index("s")
    base = wid * E_PER_W

    def step(i, carry):
        off = base + i * GCH
        pltpu.sync_copy(src_hbm.at[pl.ds(off, GCH)], idx_v)
        pltpu.sync_copy(dst_hbm.at[pl.ds(off, GCH)], idx2_v)
        cp1 = pltpu.async_copy(t_hbm.at[idx_v], rows_v, sem)
        cp2 = pltpu.async_copy(t_hbm.at[idx2_v], rows2_v, sem2)
        cp1.wait()
        pltpu.sync_copy(rows_v, gsrc_hbm.at[pl.ds(off, GCH)])
        cp2.wait()
        pltpu.sync_copy(rows2_v, gdst_hbm.at[pl.ds(off, GCH)])
        return carry

    lax.fori_loop(0, G_ITERS, step, 0)


_gather = pl.kernel(
    _gather_body,
    out_type=(jax.ShapeDtypeStruct((E, F), jnp.float32),
              jax.ShapeDtypeStruct((E, F), jnp.float32)),
    mesh=_sc_mesh,
    scratch_types=(
        pltpu.VMEM((GCH,), jnp.int32),
        pltpu.VMEM((GCH, F), jnp.float32),
        pltpu.VMEM((GCH,), jnp.int32),
        pltpu.VMEM((GCH, F), jnp.float32),
        pltpu.SemaphoreType.DMA,
        pltpu.SemaphoreType.DMA,
    ),
)


def _scatter_body(vals_hbm, idx0_hbm, idx1_hbm, zeros_hbm, agg_hbm,
                  acc_sh, vbuf, ibuf):
    c = lax.axis_index("c")
    s = lax.axis_index("s")
    pltpu.sync_copy(zeros_hbm, acc_sh.at[pl.ds(s * ZR, ZR)])
    plsc.subcore_barrier()

    base = s * E_PER_S

    def scatter_from(idx_hbm):
        def step(i, carry):
            off = base + i * SCH
            pltpu.sync_copy(idx_hbm.at[pl.ds(off, SCH)], ibuf)
            pltpu.sync_copy(vals_hbm.at[pl.ds(off, SCH)], vbuf)
            pltpu.sync_copy(vbuf, acc_sh.at[ibuf], add=True)
            return carry
        lax.fori_loop(0, S_ITERS, step, 0)

    @pl.when(c == 0)
    def _():
        scatter_from(idx0_hbm)

    @pl.when(c == 1)
    def _():
        scatter_from(idx1_hbm)

    plsc.subcore_barrier()
    out_base = c * ACC_ROWS + s * ZR
    pltpu.sync_copy(acc_sh.at[pl.ds(s * ZR, ZR)],
                    agg_hbm.at[pl.ds(out_base, ZR)])


_scatter = pl.kernel(
    _scatter_body,
    out_type=jax.ShapeDtypeStruct((2 * ACC_ROWS, VW), jnp.float32),
    mesh=_sc_mesh,
    scratch_types=(
        pltpu.VMEM_SHARED((ACC_ROWS, VW), jnp.float32),
        pltpu.VMEM((SCH, VW), jnp.float32),
        pltpu.VMEM((SCH,), jnp.int32),
    ),
)


def _edge_body(gsrc, gdst, dst3, w1s, w1d, w1q, b1, w2, b2, g, bln,
               lgw, lgb, lxw, lxb, vals, i0, i1):
    hs = gsrc[:, :D]
    xs = gsrc[:, D:D + CD]
    hd = gdst[:, :D]
    xd = gdst[:, D:D + CD]
    r = xs - xd
    dsq = jnp.sum(r * r, axis=1, keepdims=True)
    rn = r / (jnp.sqrt(dsq) + 1e-8)
    z = (jnp.dot(hs, w1s[...], preferred_element_type=jnp.float32)
         + jnp.dot(hd, w1d[...], preferred_element_type=jnp.float32)
         + dsq * w1q[...] + b1[...])
    m = _silu(z)
    m = _silu(jnp.dot(m, w2[...], preferred_element_type=jnp.float32) + b2[...])
    m = _ln(m, g[...], bln[...])
    alpha = jax.nn.sigmoid(jnp.sum(m * lgw[...], axis=1, keepdims=True) + lgb[...])
    mg = m * alpha
    cw = jnp.tanh(jnp.sum(mg * lxw[...], axis=1, keepdims=True) + lxb[...])
    cmsg = rn * cw
    one = jnp.ones((BE, 1), jnp.float32)
    zero = jnp.zeros((BE, 1), jnp.float32)
    vals[...] = jnp.concatenate([mg, cmsg, one, zero], axis=1)
    d = dst3[0, 0, :]
    i0[0, 0, :] = jnp.where(d < HALF, d, TRASH)
    i1[0, 0, :] = jnp.where(d >= HALF, d - HALF, TRASH)


def _full(shape):
    nd = len(shape)
    return pl.BlockSpec(shape, lambda i, _n=nd: (0,) * _n)


_edge_call = pl.pallas_call(
    _edge_body,
    grid=(NBLK,),
    in_specs=[
        pl.BlockSpec((BE, F), lambda i: (i, 0)),
        pl.BlockSpec((BE, F), lambda i: (i, 0)),
        pl.BlockSpec((1, 1, BE), lambda i: (i, 0, 0)),
        _full((D, D)), _full((D, D)), _full((1, D)), _full((1, D)),
        _full((D, D)), _full((1, D)), _full((1, D)), _full((1, D)),
        _full((1, D)), _full((1, 1)), _full((1, D)), _full((1, 1)),
    ],
    out_specs=[
        pl.BlockSpec((BE, VW), lambda i: (i, 0)),
        pl.BlockSpec((1, 1, BE), lambda i: (i, 0, 0)),
        pl.BlockSpec((1, 1, BE), lambda i: (i, 0, 0)),
    ],
    out_shape=[
        jax.ShapeDtypeStruct((E, VW), jnp.float32),
        jax.ShapeDtypeStruct((NBLK, 1, BE), jnp.int32),
        jax.ShapeDtypeStruct((NBLK, 1, BE), jnp.int32),
    ],
)


def _node_body(t_old, agg, wa, wb, b1, w2, b2, t_new):
    h = t_old[:, :D]
    xc = t_old[:, D:D + CD]
    agg_h = agg[:, :D]
    agg_x = agg[:, D:D + CD]
    deg = jnp.maximum(agg[:, D + CD:D + CD + 1], 1.0)
    u = _silu(jnp.dot(h, wa[...], preferred_element_type=jnp.float32)
              + jnp.dot(agg_h, wb[...], preferred_element_type=jnp.float32)
              + b1[...])
    hh = jnp.dot(u, w2[...], preferred_element_type=jnp.float32) + b2[...]
    h2 = h + hh
    x2 = xc + agg_x / deg
    pad = jnp.zeros((RN, F - D - CD), jnp.float32)
    t_new[...] = jnp.concatenate([h2, x2, pad], axis=1)


_node_call = pl.pallas_call(
    _node_body,
    grid=(N // RN,),
    in_specs=[
        pl.BlockSpec((RN, F), lambda i: (i, 0)),
        pl.BlockSpec((RN, VW),
                     lambda i: (jnp.where(i < HALF // RN, i, i + ACC_PAD // RN), 0)),
        _full((D, D)), _full((D, D)), _full((1, D)), _full((D, D)), _full((1, D)),
    ],
    out_specs=pl.BlockSpec((RN, F), lambda i: (i, 0)),
    out_shape=jax.ShapeDtypeStruct((N, F), jnp.float32),
)


def _enc_body(x, w1a, w1b, b1, w2, b2, g, bln, t0):
    coords = x[:, :CD]
    scal = x[:, CD:CD + 1]
    speed = jnp.sqrt(jnp.sum(coords * coords, axis=1, keepdims=True))
    z = jax.nn.relu(scal * w1a[...] + speed * w1b[...] + b1[...])
    h = jnp.dot(z, w2[...], preferred_element_type=jnp.float32) + b2[...]
    h = _ln(h, g[...], bln[...])
    pad = jnp.zeros((RB, F - D - CD), jnp.float32)
    t0[...] = jnp.concatenate([h, coords, pad], axis=1)


_enc_call = pl.pallas_call(
    _enc_body,
    grid=(N // RB,),
    in_specs=[
        pl.BlockSpec((RB, 3), lambda i: (i, 0)),
        _full((1, D)), _full((1, D)), _full((1, D)),
        _full((D, D)), _full((1, D)), _full((1, D)), _full((1, D)),
    ],
    out_specs=pl.BlockSpec((RB, F), lambda i: (i, 0)),
    out_shape=jax.ShapeDtypeStruct((N, F), jnp.float32),
)


def _dec_body(t, dmw1, dmb1, dmw2, dmb2, drw1, drb1, drw2, drb2, wmix, out):
    h = t[:, :D]
    xc = t[:, D:D + CD]
    u = jax.nn.relu(jnp.dot(h, dmw1[...], preferred_element_type=jnp.float32)
                    + dmb1[...])
    mag = jnp.sum(u * dmw2[...], axis=1, keepdims=True) + dmb2[...]
    nrm = jnp.sqrt(jnp.sum(xc * xc, axis=1, keepdims=True))
    dirn = xc / jnp.maximum(nrm, 1e-12)
    v = jax.nn.relu(jnp.dot(h, drw1[...], preferred_element_type=jnp.float32)
                    + drb1[...])
    res = jnp.dot(v, drw2[...], preferred_element_type=jnp.float32) + drb2[...]
    w = wmix[0, 0]
    out[...] = w * (mag * dirn) + (1.0 - w) * res


_dec_call = pl.pallas_call(
    _dec_body,
    grid=(N // RB,),
    in_specs=[
        pl.BlockSpec((RB, F), lambda i: (i, 0)),
        _full((D, D)), _full((1, D)), _full((1, D)), _full((1, 1)),
        _full((D, D)), _full((1, D)), _full((D, CD)), _full((1, CD)),
        _full((1, 1)),
    ],
    out_specs=pl.BlockSpec((RB, CD), lambda i: (i, 0)),
    out_shape=jax.ShapeDtypeStruct((N, CD), jnp.float32),
)


def kernel(x, edge_index, enc_W1, enc_b1, enc_W2, enc_b2, enc_g, enc_bln,
           le_W1, le_b1, le_W2, le_b2, le_g, le_bln, lh_W1, lh_b1, lh_W2,
           lh_b2, lx_W, lx_b, lg_W, lg_b, dm_W1, dm_b1, dm_W2, dm_b2,
           dr_W1, dr_b1, dr_W2, dr_b2, mix):
    src = edge_index[0]
    dst = edge_index[1]
    dst3 = dst.reshape(NBLK, 1, BE)
    zrows = jnp.zeros((ZR, VW), jnp.float32)

    t = _enc_call(x,
                  enc_W1[0:1, :], enc_W1[1:2, :], enc_b1.reshape(1, D),
                  enc_W2, enc_b2.reshape(1, D),
                  enc_g.reshape(1, D), enc_bln.reshape(1, D))

    for l in range(L):
        gsrc, gdst = _gather(t, src, dst)
        vals, i0, i1 = _edge_call(
            gsrc, gdst, dst3,
            le_W1[l, :D, :], le_W1[l, D:2 * D, :], le_W1[l, 2 * D:, :],
            le_b1[l].reshape(1, D), le_W2[l], le_b2[l].reshape(1, D),
            le_g[l].reshape(1, D), le_bln[l].reshape(1, D),
            lg_W[l].reshape(1, D), lg_b[l].reshape(1, 1),
            lx_W[l].reshape(1, D), lx_b[l].reshape(1, 1))
        agg = _scatter(vals, i0.reshape(E), i1.reshape(E), zrows)
        t = _node_call(t, agg,
                       lh_W1[l, :D, :], lh_W1[l, D:, :],
                       lh_b1[l].reshape(1, D), lh_W2[l],
                       lh_b2[l].reshape(1, D))

    out = _dec_call(t,
                    dm_W1, dm_b1.reshape(1, D), dm_W2.reshape(1, D),
                    dm_b2.reshape(1, 1),
                    dr_W1, dr_b1.reshape(1, D), dr_W2, dr_b2.reshape(1, CD),
                    jax.nn.sigmoid(mix).reshape(1, 1))
    return out


# speedup vs baseline: 1.5747x; 1.5747x over previous
import jax
import jax.numpy as jnp
from jax import lax
from jax.experimental import pallas as pl
from jax.experimental.pallas import tpu as pltpu
from jax.experimental.pallas import tpu_sc as plsc

N = 50000
E = 800000
D = 64
L = 6
CD = 2

F = 128
VW = 68

NC = 2
NS = 16
NW = NC * NS

HALF = N // 2
ACC_PAD = 600
ACC_ROWS = HALF + ACC_PAD
TRASH = HALF
ZR = ACC_ROWS // NS

E_PER_W = E // NW
GCH = 128
G_FULL = E_PER_W // GCH
G_TAIL = E_PER_W - G_FULL * GCH

E_PER_S = E // NS
SCH = 80
S_ITERS = E_PER_S // SCH

BE = 4000
NBLK = E // BE
RN = 200
RB = 2000


def _silu(v):
    return v * jax.nn.sigmoid(v)


def _ln(v, g, b):
    m = v.mean(axis=-1, keepdims=True)
    var = ((v - m) ** 2).mean(axis=-1, keepdims=True)
    return (v - m) / jnp.sqrt(var + 1e-5) * g + b


def _gather_body(t_hbm, src_hbm, dst_hbm, gsrc_hbm, gdst_hbm,
                 idx_v, rows_v, idx2_v, rows2_v, sem, sem2):
    wid = lax.axis_index("c") * NS + lax.axis_index("s")
    base = wid * E_PER_W

    def step(i, carry):
        off = base + i * GCH
        pltpu.sync_copy(src_hbm.at[pl.ds(off, GCH)], idx_v)
        pltpu.sync_copy(dst_hbm.at[pl.ds(off, GCH)], idx2_v)
        cp1 = pltpu.async_copy(t_hbm.at[idx_v], rows_v, sem)
        cp2 = pltpu.async_copy(t_hbm.at[idx2_v], rows2_v, sem2)
        cp1.wait()
        pltpu.sync_copy(rows_v, gsrc_hbm.at[pl.ds(off, GCH)])
        cp2.wait()
        pltpu.sync_copy(rows2_v, gdst_hbm.at[pl.ds(off, GCH)])
        return carry

    lax.fori_loop(0, G_FULL, step, 0)

    toff = base + G_FULL * GCH
    idx_t = idx_v.at[pl.ds(0, G_TAIL)]
    idx2_t = idx2_v.at[pl.ds(0, G_TAIL)]
    rows_t = rows_v.at[pl.ds(0, G_TAIL)]
    rows2_t = rows2_v.at[pl.ds(0, G_TAIL)]
    pltpu.sync_copy(src_hbm.at[pl.ds(toff, G_TAIL)], idx_t)
    pltpu.sync_copy(dst_hbm.at[pl.ds(toff, G_TAIL)], idx2_t)
    cp1 = pltpu.async_copy(t_hbm.at[idx_t], rows_t, sem)
    cp2 = pltpu.async_copy(t_hbm.at[idx2_t], rows2_t, sem2)
    cp1.wait()
    pltpu.sync_copy(rows_t, gsrc_hbm.at[pl.ds(toff, G_TAIL)])
    cp2.wait()
    pltpu.sync_copy(rows2_t, gdst_hbm.at[pl.ds(toff, G_TAIL)])


import functools


@functools.cache
def _sc_mesh():
    return plsc.VectorSubcoreMesh(
        core_axis_name="c", subcore_axis_name="s",
        num_cores=NC, num_subcores=NS)


@functools.cache
def _gather_kernel():
    return pl.kernel(
        _gather_body,
        out_type=(jax.ShapeDtypeStruct((E, F), jnp.float32),
                  jax.ShapeDtypeStruct((E, F), jnp.float32)),
        mesh=_sc_mesh(),
        scratch_types=(
            pltpu.VMEM((GCH,), jnp.int32),
            pltpu.VMEM((GCH, F), jnp.float32),
            pltpu.VMEM((GCH,), jnp.int32),
            pltpu.VMEM((GCH, F), jnp.float32),
            pltpu.SemaphoreType.DMA,
            pltpu.SemaphoreType.DMA,
        ),
    )


def _gather(t, src, dst):
    return _gather_kernel()(t, src, dst)


def _scatter_body(vals_hbm, idx0_hbm, idx1_hbm, zeros_hbm, agg_hbm,
                  acc_sh, vbuf, ibuf):
    c = lax.axis_index("c")
    s = lax.axis_index("s")
    pltpu.sync_copy(zeros_hbm, vbuf)

    def zstep(k, carry):
        pltpu.sync_copy(vbuf, acc_sh.at[pl.ds(s * ZR + k * SCH, SCH)])
        return carry
    lax.fori_loop(0, ZR // SCH, zstep, 0)
    plsc.subcore_barrier()

    base = s * E_PER_S

    def scatter_from(idx_hbm):
        def step(i, carry):
            off = base + i * SCH
            pltpu.sync_copy(idx_hbm.at[pl.ds(off, SCH)], ibuf)
            pltpu.sync_copy(vals_hbm.at[pl.ds(off, SCH)], vbuf)
            pltpu.sync_copy(vbuf, acc_sh.at[ibuf], add=True)
            return carry
        lax.fori_loop(0, S_ITERS, step, 0)

    if False:
        @pl.when(c == 0)
        def _():
            scatter_from(idx0_hbm)

        @pl.when(c == 1)
        def _():
            scatter_from(idx1_hbm)

    plsc.subcore_barrier()
    out_base = c * ACC_ROWS + s * ZR

    def ostep(k, carry):
        pltpu.sync_copy(acc_sh.at[pl.ds(s * ZR + k * SCH, SCH)], vbuf)
        pltpu.sync_copy(vbuf, agg_hbm.at[pl.ds(out_base + k * SCH, SCH)])
        return carry
    lax.fori_loop(0, ZR // SCH, ostep, 0)


@functools.cache
def _scatter_kernel():
    return pl.kernel(
        _scatter_body,
        out_type=jax.ShapeDtypeStruct((2 * ACC_ROWS, VW), jnp.float32),
        mesh=_sc_mesh(),
        scratch_types=(
            pltpu.VMEM_SHARED((ACC_ROWS, VW), jnp.float32),
            pltpu.VMEM((SCH, VW), jnp.float32),
            pltpu.VMEM((SCH,), jnp.int32),
        ),
    )


def _scatter(vals, i0, i1, zrows):
    acc0 = jnp.zeros((ACC_ROWS, VW), jnp.float32).at[i0].add(vals)
    acc1 = jnp.zeros((ACC_ROWS, VW), jnp.float32).at[i1].add(vals)
    return jnp.concatenate([acc0, acc1], axis=0)


def _edge_body(gsrc, gdst, dst3, w1, b1, w2, b2, g, bln,
               lgw, lgb, lxw, lxb, vals, i0, i1):
    hs = gsrc[:, :D]
    xs = gsrc[:, D:D + CD]
    hd = gdst[:, :D]
    xd = gdst[:, D:D + CD]
    r = xs - xd
    dsq = jnp.sum(r * r, axis=1, keepdims=True)
    rn = r / (jnp.sqrt(dsq) + 1e-8)
    min_ = jnp.concatenate([hs, hd, dsq], axis=1)
    z = jnp.dot(min_, w1[...], preferred_element_type=jnp.float32) + b1[...]
    m = _silu(z)
    m = _silu(jnp.dot(m, w2[...], preferred_element_type=jnp.float32) + b2[...])
    m = _ln(m, g[...], bln[...])
    alpha = jax.nn.sigmoid(
        jnp.dot(m, lgw[...], preferred_element_type=jnp.float32) + lgb[...])
    mg = m * alpha
    cw = jnp.tanh(
        jnp.dot(mg, lxw[...], preferred_element_type=jnp.float32) + lxb[...])
    cmsg = rn * cw
    one = jnp.ones((BE, 1), jnp.float32)
    zero = jnp.zeros((BE, 1), jnp.float32)
    vals[...] = jnp.concatenate([mg, cmsg, one, zero], axis=1)
    d = dst3[0, 0, :]
    i0[0, 0, :] = jnp.where(d < HALF, d, TRASH)
    i1[0, 0, :] = jnp.where(d >= HALF, d - HALF, TRASH)


def _full(shape):
    nd = len(shape)
    return pl.BlockSpec(shape, lambda i, _n=nd: (0,) * _n)


_edge_call = pl.pallas_call(
    _edge_body,
    grid=(NBLK,),
    in_specs=[
        pl.BlockSpec((BE, F), lambda i: (i, 0)),
        pl.BlockSpec((BE, F), lambda i: (i, 0)),
        pl.BlockSpec((1, 1, BE), lambda i: (i, 0, 0)),
        _full((2 * D + 1, D)), _full((1, D)),
        _full((D, D)), _full((1, D)), _full((1, D)), _full((1, D)),
        _full((D, 1)), _full((1, 1)), _full((D, 1)), _full((1, 1)),
    ],
    out_specs=[
        pl.BlockSpec((BE, VW), lambda i: (i, 0)),
        pl.BlockSpec((1, 1, BE), lambda i: (i, 0, 0)),
        pl.BlockSpec((1, 1, BE), lambda i: (i, 0, 0)),
    ],
    out_shape=[
        jax.ShapeDtypeStruct((E, VW), jnp.float32),
        jax.ShapeDtypeStruct((NBLK, 1, BE), jnp.int32),
        jax.ShapeDtypeStruct((NBLK, 1, BE), jnp.int32),
    ],
)


def _node_body(t_old, agg, w1, b1, w2, b2, t_new):
    h = t_old[:, :D]
    xc = t_old[:, D:D + CD]
    agg_h = agg[:, :D]
    agg_x = agg[:, D:D + CD]
    deg = jnp.maximum(agg[:, D + CD:D + CD + 1], 1.0)
    hin = jnp.concatenate([h, agg_h], axis=1)
    u = _silu(jnp.dot(hin, w1[...], preferred_element_type=jnp.float32)
              + b1[...])
    hh = jnp.dot(u, w2[...], preferred_element_type=jnp.float32) + b2[...]
    h2 = h + hh
    x2 = xc + agg_x / deg
    pad = jnp.zeros((RN, F - D - CD), jnp.float32)
    t_new[...] = jnp.concatenate([h2, x2, pad], axis=1)


_node_call = pl.pallas_call(
    _node_body,
    grid=(N // RN,),
    in_specs=[
        pl.BlockSpec((RN, F), lambda i: (i, 0)),
        pl.BlockSpec((RN, VW),
                     lambda i: (jnp.where(i < HALF // RN, i, i + ACC_PAD // RN), 0)),
        _full((2 * D, D)), _full((1, D)), _full((D, D)), _full((1, D)),
    ],
    out_specs=pl.BlockSpec((RN, F), lambda i: (i, 0)),
    out_shape=jax.ShapeDtypeStruct((N, F), jnp.float32),
)


def _enc_body(x, w1, b1, w2, b2, g, bln, t0):
    coords = x[:, :CD]
    scal = x[:, CD:CD + 1]
    speed = jnp.sqrt(jnp.sum(coords * coords, axis=1, keepdims=True))
    hin = jnp.concatenate([scal, speed], axis=1)
    z = jax.nn.relu(
        jnp.dot(hin, w1[...], preferred_element_type=jnp.float32) + b1[...])
    h = jnp.dot(z, w2[...], preferred_element_type=jnp.float32) + b2[...]
    h = _ln(h, g[...], bln[...])
    pad = jnp.zeros((RB, F - D - CD), jnp.float32)
    t0[...] = jnp.concatenate([h, coords, pad], axis=1)


_enc_call = pl.pallas_call(
    _enc_body,
    grid=(N // RB,),
    in_specs=[
        pl.BlockSpec((RB, 3), lambda i: (i, 0)),
        _full((2, D)), _full((1, D)),
        _full((D, D)), _full((1, D)), _full((1, D)), _full((1, D)),
    ],
    out_specs=pl.BlockSpec((RB, F), lambda i: (i, 0)),
    out_shape=jax.ShapeDtypeStruct((N, F), jnp.float32),
)


def _dec_body(t, dmw1, dmb1, dmw2, dmb2, drw1, drb1, drw2, drb2, wmix, out):
    h = t[:, :D]
    xc = t[:, D:D + CD]
    u = jax.nn.relu(jnp.dot(h, dmw1[...], preferred_element_type=jnp.float32)
                    + dmb1[...])
    mag = jnp.dot(u, dmw2[...], preferred_element_type=jnp.float32) + dmb2[...]
    nrm = jnp.sqrt(jnp.sum(xc * xc, axis=1, keepdims=True))
    dirn = xc / jnp.maximum(nrm, 1e-12)
    v = jax.nn.relu(jnp.dot(h, drw1[...], preferred_element_type=jnp.float32)
                    + drb1[...])
    res = jnp.dot(v, drw2[...], preferred_element_type=jnp.float32) + drb2[...]
    w = wmix[0, 0]
    out[...] = w * (mag * dirn) + (1.0 - w) * res


_dec_call = pl.pallas_call(
    _dec_body,
    grid=(N // RB,),
    in_specs=[
        pl.BlockSpec((RB, F), lambda i: (i, 0)),
        _full((D, D)), _full((1, D)), _full((D, 1)), _full((1, 1)),
        _full((D, D)), _full((1, D)), _full((D, CD)), _full((1, CD)),
        _full((1, 1)),
    ],
    out_specs=pl.BlockSpec((RB, CD), lambda i: (i, 0)),
    out_shape=jax.ShapeDtypeStruct((N, CD), jnp.float32),
)


def kernel(x, edge_index, enc_W1, enc_b1, enc_W2, enc_b2, enc_g, enc_bln,
           le_W1, le_b1, le_W2, le_b2, le_g, le_bln, lh_W1, lh_b1, lh_W2,
           lh_b2, lx_W, lx_b, lg_W, lg_b, dm_W1, dm_b1, dm_W2, dm_b2,
           dr_W1, dr_b1, dr_W2, dr_b2, mix):
    src = edge_index[0]
    dst = edge_index[1]
    dst3 = dst.reshape(NBLK, 1, BE)
    zrows = jnp.zeros((SCH, VW), jnp.float32)

    t = _enc_call(x,
                  enc_W1, enc_b1.reshape(1, D),
                  enc_W2, enc_b2.reshape(1, D),
                  enc_g.reshape(1, D), enc_bln.reshape(1, D))

    for l in range(L):
        gsrc, gdst = _gather(t, src, dst)
        vals, i0, i1 = _edge_call(
            gsrc, gdst, dst3,
            le_W1[l],
            le_b1[l].reshape(1, D), le_W2[l], le_b2[l].reshape(1, D),
            le_g[l].reshape(1, D), le_bln[l].reshape(1, D),
            lg_W[l], lg_b[l].reshape(1, 1),
            lx_W[l], lx_b[l].reshape(1, 1))
        agg = _scatter(vals, i0.reshape(E), i1.reshape(E), zrows)
        t = _node_call(t, agg,
                       lh_W1[l],
                       lh_b1[l].reshape(1, D), lh_W2[l],
                       lh_b2[l].reshape(1, D))

    out = _dec_call(t,
                    dm_W1, dm_b1.reshape(1, D), dm_W2,
                    dm_b2.reshape(1, 1),
                    dr_W1, dr_b1.reshape(1, D), dr_W2, dr_b2.reshape(1, CD),
                    jax.nn.sigmoid(mix).reshape(1, 1))
    return out
